# baseline (speedup 1.0000x reference)
import jax
import jax.numpy as jnp
from jax import lax
from jax.experimental import pallas as pl
from jax.experimental.pallas import tpu as pltpu
from jax.experimental.pallas import tpu_sc as plsc

B, H, W = 8, 384, 512
HW = H * W
NUM_TOP = 16384

NC, NS = 2, 16
NW = NC * NS
CHUNKS_PER_B = NW // B
CHUNK = NUM_TOP // CHUNKS_PER_B
ROWS = CHUNK // 128
LANES = 16
VITERS = 128 // LANES
OCHUNKS = 4
OROWS = ROWS // OCHUNKS
OLEN = CHUNK // OCHUNKS


def _sc_body(depth_hbm, coeff_hbm, sidx_hbm, ridx_hbm, out_hbm,
             sidx_v, ridx_v, d_v, coeff_v, out_v,
             gsem0, gsem1, gsem2, gsem3, rsem, osem):
    c = lax.axis_index("c")
    s = lax.axis_index("s")
    wid = s * NC + c
    b = wid // CHUNKS_PER_B
    ch = lax.rem(wid, CHUNKS_PER_B)
    base = ch * CHUNK
    gsems = [gsem0, gsem1, gsem2, gsem3]

    pltpu.sync_copy(sidx_hbm.at[b, pl.ds(base, CHUNK)], sidx_v)
    gathers = [
        pltpu.async_copy(depth_hbm.at[sidx_v.at[pl.ds(j * 128, 128)]],
                         d_v.at[pl.ds(j * 128, 128)], gsems[j // OROWS])
        for j in range(ROWS)
    ]
    rcopy = pltpu.async_copy(ridx_hbm.at[b, pl.ds(base, CHUNK)], ridx_v, rsem)
    pltpu.sync_copy(coeff_hbm.at[b], coeff_v)

    ones = jnp.full((LANES,), 1.0, dtype=jnp.float32)

    def ones_body(j, carry):
        for l in range(VITERS):
            out_v[3, pl.ds(j * 128 + l * LANES, LANES)] = ones
        return carry

    lax.fori_loop(0, ROWS, ones_body, 0)

    k00 = coeff_v[0]
    k01 = coeff_v[1]
    k02 = coeff_v[2]
    k10 = coeff_v[3]
    k11 = coeff_v[4]
    k12 = coeff_v[5]
    k20 = coeff_v[6]
    k21 = coeff_v[7]
    k22 = coeff_v[8]

    rcopy.wait()
    out_dmas = []
    for oc in range(OCHUNKS):
        for j in range(OROWS):
            gathers[oc * OROWS + j].wait()

        def row_body(j, carry):
            for l in range(VITERS):
                off = oc * OLEN + j * 128 + l * LANES
                r = ridx_v[pl.ds(off, LANES)]
                d = d_v[pl.ds(off, LANES)]
                x = (r & (W - 1)).astype(jnp.float32)
                y = (r >> 9).astype(jnp.float32)
                out_v[0, pl.ds(off, LANES)] = d * (k00 * x + k01 * y + k02)
                out_v[1, pl.ds(off, LANES)] = d * (k10 * x + k11 * y + k12)
                out_v[2, pl.ds(off, LANES)] = d * (k20 * x + k21 * y + k22)
            return carry

        lax.fori_loop(0, OROWS, row_body, 0)
        out_dmas.append(pltpu.async_copy(
            out_v.at[:, pl.ds(oc * OLEN, OLEN)],
            out_hbm.at[b, :, pl.ds(base + oc * OLEN, OLEN)], osem))

    for dma in out_dmas:
        dma.wait()


@jax.jit
def _backproject(depth, inv_K, top_k_indices):
    depth_lin = depth.reshape(B, H // 8, 8, W // 128, 128)
    depth_lin = depth_lin.transpose(0, 1, 3, 2, 4).reshape(B * HW)

    base = (jnp.arange(B, dtype=jnp.int32) * HW)[:, None]
    g = top_k_indices + base
    sidx = ((g & ~4095) | ((g & 0x180) << 3) | ((g & 0xE00) >> 2) | (g & 127))

    A = inv_K[:, :3, :3]
    coeff = A.reshape(B, 9)
    coeff16 = jnp.broadcast_to(coeff[:, :, None], (B, 9, LANES))

    run = pl.kernel(
        _sc_body,
        out_type=jax.ShapeDtypeStruct((B, 4, NUM_TOP), jnp.float32),
        mesh=plsc.VectorSubcoreMesh(core_axis_name="c", subcore_axis_name="s"),
        scratch_types=[
            pltpu.VMEM((CHUNK,), jnp.int32),
            pltpu.VMEM((CHUNK,), jnp.int32),
            pltpu.VMEM((CHUNK,), jnp.float32),
            pltpu.VMEM((9, LANES), jnp.float32),
            pltpu.VMEM((4, CHUNK), jnp.float32),
            pltpu.SemaphoreType.DMA,
            pltpu.SemaphoreType.DMA,
            pltpu.SemaphoreType.DMA,
            pltpu.SemaphoreType.DMA,
            pltpu.SemaphoreType.DMA,
            pltpu.SemaphoreType.DMA,
        ],
    )
    return run(depth_lin, coeff16, sidx, top_k_indices)


def kernel(depth, inv_K, top_k_indices):
    return _backproject(depth, inv_K, top_k_indices)


# speedup vs baseline: 1.0116x; 1.0116x over previous
import jax
import jax.numpy as jnp
from jax import lax
from jax.experimental import pallas as pl
from jax.experimental.pallas import tpu as pltpu
from jax.experimental.pallas import tpu_sc as plsc

B, H, W = 8, 384, 512
HW = H * W
NUM_TOP = 16384

NC, NS = 2, 16
NW = NC * NS
CHUNKS_PER_B = NW // B
CHUNK = NUM_TOP // CHUNKS_PER_B
ROWS = CHUNK // 128
LANES = 16
VITERS = 128 // LANES
OCHUNKS = 4
OROWS = ROWS // OCHUNKS
OLEN = CHUNK // OCHUNKS


def _sc_body(depth_hbm, coeff_hbm, ridx_hbm, out_hbm,
             sidx_v, ridx_v, d_v, coeff_v, out_v, gsem, osem):
    c = lax.axis_index("c")
    s = lax.axis_index("s")
    wid = s * NC + c
    b = wid // CHUNKS_PER_B
    ch = lax.rem(wid, CHUNKS_PER_B)
    base = ch * CHUNK
    bhw = b * HW

    pltpu.sync_copy(ridx_hbm.at[b, pl.ds(base, CHUNK)], ridx_v)

    def swiz_body(j, carry):
        for l in range(VITERS):
            off = j * 128 + l * LANES
            r = ridx_v[pl.ds(off, LANES)]
            sidx_v[pl.ds(off, LANES)] = (
                (r & ~4095) | ((r & 0x180) << 3) | ((r & 0xE00) >> 2)
                | (r & 127)) + bhw
        pltpu.async_copy(depth_hbm.at[sidx_v.at[pl.ds(j * 128, 128)]],
                         d_v.at[pl.ds(j * 128, 128)], gsem)
        return carry

    lax.fori_loop(0, ROWS, swiz_body, 0)

    pltpu.sync_copy(coeff_hbm.at[b], coeff_v)

    k00 = coeff_v[0]
    k01 = coeff_v[1]
    k02 = coeff_v[2]
    k10 = coeff_v[3]
    k11 = coeff_v[4]
    k12 = coeff_v[5]
    k20 = coeff_v[6]
    k21 = coeff_v[7]
    k22 = coeff_v[8]
    ones = jnp.full((LANES,), 1.0, dtype=jnp.float32)

    pltpu.make_async_copy(depth_hbm.at[pl.ds(0, CHUNK)], d_v, gsem).wait()

    out_dmas = []
    for oc in range(OCHUNKS):

        def row_body(j, carry):
            for l in range(VITERS):
                off = oc * OLEN + j * 128 + l * LANES
                r = ridx_v[pl.ds(off, LANES)]
                d = d_v[pl.ds(off, LANES)]
                x = (r & (W - 1)).astype(jnp.float32)
                y = (r >> 9).astype(jnp.float32)
                out_v[0, pl.ds(off, LANES)] = d * (k00 * x + k01 * y + k02)
                out_v[1, pl.ds(off, LANES)] = d * (k10 * x + k11 * y + k12)
                out_v[2, pl.ds(off, LANES)] = d * (k20 * x + k21 * y + k22)
                out_v[3, pl.ds(off, LANES)] = ones
            return carry

        lax.fori_loop(0, OROWS, row_body, 0)
        out_dmas.append(pltpu.async_copy(
            out_v.at[:, pl.ds(oc * OLEN, OLEN)],
            out_hbm.at[b, :, pl.ds(base + oc * OLEN, OLEN)], osem))

    for dma in out_dmas:
        dma.wait()


@jax.jit
def _backproject(depth, inv_K, top_k_indices):
    depth_lin = depth.reshape(B, H // 8, 8, W // 128, 128)
    depth_lin = depth_lin.transpose(0, 1, 3, 2, 4).reshape(B * HW)

    A = inv_K[:, :3, :3]
    coeff = A.reshape(B, 9)
    coeff16 = jnp.broadcast_to(coeff[:, :, None], (B, 9, LANES))

    run = pl.kernel(
        _sc_body,
        out_type=jax.ShapeDtypeStruct((B, 4, NUM_TOP), jnp.float32),
        mesh=plsc.VectorSubcoreMesh(core_axis_name="c", subcore_axis_name="s"),
        scratch_types=[
            pltpu.VMEM((CHUNK,), jnp.int32),
            pltpu.VMEM((CHUNK,), jnp.int32),
            pltpu.VMEM((CHUNK,), jnp.float32),
            pltpu.VMEM((9, LANES), jnp.float32),
            pltpu.VMEM((4, CHUNK), jnp.float32),
            pltpu.SemaphoreType.DMA,
            pltpu.SemaphoreType.DMA,
        ],
    )
    return run(depth_lin, coeff16, top_k_indices)


def kernel(depth, inv_K, top_k_indices):
    return _backproject(depth, inv_K, top_k_indices)


# speedup vs baseline: 1.0404x; 1.0285x over previous
import jax
import jax.numpy as jnp
from jax import lax
from jax.experimental import pallas as pl
from jax.experimental.pallas import tpu as pltpu
from jax.experimental.pallas import tpu_sc as plsc

B, H, W = 8, 384, 512
HW = H * W
NUM_TOP = 16384

NC, NS = 2, 16
NW = NC * NS
CHUNKS_PER_B = NW // B
CHUNK = NUM_TOP // CHUNKS_PER_B
ROWS = CHUNK // 128
LANES = 16
VITERS = 128 // LANES
OCHUNKS = 2
OROWS = ROWS // OCHUNKS
OLEN = CHUNK // OCHUNKS


def _sc_body(depth_hbm, coeff_hbm, ridx_hbm, out_hbm,
             sidx_v, ridx_v, d_v, coeff_v, out_v, gsem, osem):
    c = lax.axis_index("c")
    s = lax.axis_index("s")
    wid = s * NC + c
    b = wid // CHUNKS_PER_B
    ch = lax.rem(wid, CHUNKS_PER_B)
    base = ch * CHUNK
    bhw = b * HW

    pltpu.sync_copy(ridx_hbm.at[b, pl.ds(base, CHUNK)], ridx_v)

    def swiz_body(j, carry):
        for l in range(VITERS):
            off = j * 128 + l * LANES
            r = ridx_v[pl.ds(off, LANES)]
            sidx_v[pl.ds(off, LANES)] = (
                (r & ~4095) | ((r & 0x180) << 3) | ((r & 0xE00) >> 2)
                | (r & 127)) + bhw
        pltpu.async_copy(depth_hbm.at[sidx_v.at[pl.ds(j * 128, 128)]],
                         d_v.at[pl.ds(j * 128, 128)], gsem)
        return carry

    lax.fori_loop(0, ROWS, swiz_body, 0)

    pltpu.sync_copy(coeff_hbm.at[b], coeff_v)

    k00 = coeff_v[0]
    k01 = coeff_v[1]
    k02 = coeff_v[2]
    k10 = coeff_v[3]
    k11 = coeff_v[4]
    k12 = coeff_v[5]
    k20 = coeff_v[6]
    k21 = coeff_v[7]
    k22 = coeff_v[8]
    ones = jnp.full((LANES,), 1.0, dtype=jnp.float32)

    pltpu.make_async_copy(depth_hbm.at[pl.ds(0, CHUNK)], d_v, gsem).wait()

    out_dmas = []
    for oc in range(OCHUNKS):

        def row_body(j, carry):
            for l in range(VITERS):
                off = oc * OLEN + j * 128 + l * LANES
                r = ridx_v[pl.ds(off, LANES)]
                d = d_v[pl.ds(off, LANES)]
                x = (r & (W - 1)).astype(jnp.float32)
                y = (r >> 9).astype(jnp.float32)
                out_v[0, pl.ds(off, LANES)] = d * (k00 * x + k01 * y + k02)
                out_v[1, pl.ds(off, LANES)] = d * (k10 * x + k11 * y + k12)
                out_v[2, pl.ds(off, LANES)] = d * (k20 * x + k21 * y + k22)
                out_v[3, pl.ds(off, LANES)] = ones
            return carry

        lax.fori_loop(0, OROWS, row_body, 0)
        out_dmas.append(pltpu.async_copy(
            out_v.at[:, pl.ds(oc * OLEN, OLEN)],
            out_hbm.at[b, :, pl.ds(base + oc * OLEN, OLEN)], osem))

    for dma in out_dmas:
        dma.wait()


@jax.jit
def _backproject(depth, inv_K, top_k_indices):
    depth_lin = depth.reshape(B, H // 8, 8, W // 128, 128)
    depth_lin = depth_lin.transpose(0, 1, 3, 2, 4).reshape(B * HW)

    A = inv_K[:, :3, :3]
    coeff = A.reshape(B, 9)
    coeff16 = jnp.broadcast_to(coeff[:, :, None], (B, 9, LANES))

    run = pl.kernel(
        _sc_body,
        out_type=jax.ShapeDtypeStruct((B, 4, NUM_TOP), jnp.float32),
        mesh=plsc.VectorSubcoreMesh(core_axis_name="c", subcore_axis_name="s"),
        scratch_types=[
            pltpu.VMEM((CHUNK,), jnp.int32),
            pltpu.VMEM((CHUNK,), jnp.int32),
            pltpu.VMEM((CHUNK,), jnp.float32),
            pltpu.VMEM((9, LANES), jnp.float32),
            pltpu.VMEM((4, CHUNK), jnp.float32),
            pltpu.SemaphoreType.DMA,
            pltpu.SemaphoreType.DMA,
        ],
    )
    return run(depth_lin, coeff16, top_k_indices)


def kernel(depth, inv_K, top_k_indices):
    return _backproject(depth, inv_K, top_k_indices)
